# Initial kernel scaffold; baseline (speedup 1.0000x reference)
import jax
import jax.numpy as jnp
from jax.experimental import pallas as pl


def kernel(x, edge_index, params):
    raise NotImplementedError("write your pallas kernel here")


# speedup vs baseline: 2.3516x; 2.3516x over previous
import functools

import jax
import jax.numpy as jnp
from jax import lax
from jax.experimental import pallas as pl
from jax.experimental.pallas import tpu as pltpu
from jax.experimental.pallas import tpu_sc as plsc

N = 10000
E = 320000
NP = 10240
WB = 128
BN = 256
N_TILES = 16
EDGES_PER_TILE = E // N_TILES
CHUNK = 128
N_FULL = EDGES_PER_TILE // CHUNK
TAIL = EDGES_PER_TILE - N_FULL * CHUNK
ROWS_PER_TILE = NP // N_TILES


def _pad128(d):
    return ((d + 127) // 128) * 128


@functools.lru_cache(maxsize=None)
def _make_agg(nwb):
    mesh = plsc.VectorSubcoreMesh(core_axis_name="c", subcore_axis_name="s")

    @functools.partial(
        pl.kernel,
        mesh=mesh,
        out_type=jax.ShapeDtypeStruct((nwb, NP, WB), jnp.float32),
        scratch_types=[
            pltpu.VMEM((CHUNK,), jnp.int32),
            pltpu.VMEM((CHUNK,), jnp.int32),
            pltpu.VMEM((CHUNK, WB), jnp.float32),
            pltpu.VMEM((TAIL,), jnp.int32),
            pltpu.VMEM((TAIL,), jnp.int32),
            pltpu.VMEM((TAIL, WB), jnp.float32),
            pltpu.VMEM_SHARED((NP, WB), jnp.float32),
            pltpu.SemaphoreType.DMA,
        ],
    )
    def agg(p_hbm, src_hbm, dst_hbm, zeros_hbm, out_hbm,
            src_v, dst_v, rows_v, src_t, dst_t, rows_t, acc_sh, sem):
        c = lax.axis_index("c")
        s = lax.axis_index("s")
        row0 = s * ROWS_PER_TILE
        ebase = s * EDGES_PER_TILE

        for wj in range(nwb):
            @pl.when(c == (wj % 2))
            def _():
                pltpu.sync_copy(zeros_hbm.at[pl.ds(row0, ROWS_PER_TILE)],
                                acc_sh.at[pl.ds(row0, ROWS_PER_TILE)])
                plsc.subcore_barrier()

                def chunk(k, carry):
                    base = ebase + k * CHUNK
                    pltpu.sync_copy(src_hbm.at[pl.ds(base, CHUNK)], src_v)
                    pltpu.sync_copy(dst_hbm.at[pl.ds(base, CHUNK)], dst_v)
                    pltpu.async_copy(p_hbm.at[wj].at[src_v], rows_v, sem).wait()
                    pltpu.sync_copy(rows_v, acc_sh.at[dst_v], add=True)
                    return carry

                lax.fori_loop(0, N_FULL, chunk, 0)

                tbase = ebase + N_FULL * CHUNK
                pltpu.sync_copy(src_hbm.at[pl.ds(tbase, TAIL)], src_t)
                pltpu.sync_copy(dst_hbm.at[pl.ds(tbase, TAIL)], dst_t)
                pltpu.async_copy(p_hbm.at[wj].at[src_t], rows_t, sem).wait()
                pltpu.sync_copy(rows_t, acc_sh.at[dst_t], add=True)

                plsc.subcore_barrier()
                pltpu.sync_copy(acc_sh.at[pl.ds(row0, ROWS_PER_TILE)],
                                out_hbm.at[wj].at[pl.ds(row0, ROWS_PER_TILE)])
                plsc.subcore_barrier()

    return agg


def _proj(h, wlt):
    din = h.shape[1]
    nwb = wlt.shape[1] // WB

    def body(h_ref, w_ref, o_ref):
        o_ref[...] = jnp.dot(h_ref[...], w_ref[...],
                             preferred_element_type=jnp.float32)[None]

    return pl.pallas_call(
        body,
        grid=(NP // BN, nwb),
        in_specs=[
            pl.BlockSpec((BN, din), lambda i, j: (i, 0)),
            pl.BlockSpec((din, WB), lambda i, j: (0, j)),
        ],
        out_specs=pl.BlockSpec((1, BN, WB), lambda i, j: (j, i, 0)),
        out_shape=jax.ShapeDtypeStruct((nwb, NP, WB), jnp.float32),
    )(h, wlt)


def _combine(m_blk, cnt, h, wrt, b):
    din = h.shape[1]
    dout = wrt.shape[1]
    nwb = dout // WB

    def body(m_ref, c_ref, h_ref, w_ref, b_ref, o_ref):
        recip = 1.0 / jnp.maximum(c_ref[...], 1.0)
        acc = m_ref[0] * recip + b_ref[...]
        acc += jnp.dot(h_ref[...], w_ref[...], preferred_element_type=jnp.float32)
        o_ref[...] = jnp.maximum(acc, 0.0)

    return pl.pallas_call(
        body,
        grid=(NP // BN, nwb),
        in_specs=[
            pl.BlockSpec((1, BN, WB), lambda i, j: (j, i, 0)),
            pl.BlockSpec((BN, 1), lambda i, j: (i, 0)),
            pl.BlockSpec((BN, din), lambda i, j: (i, 0)),
            pl.BlockSpec((din, WB), lambda i, j: (0, j)),
            pl.BlockSpec((1, WB), lambda i, j: (0, j)),
        ],
        out_specs=pl.BlockSpec((BN, WB), lambda i, j: (i, j)),
        out_shape=jax.ShapeDtypeStruct((NP, dout), jnp.float32),
    )(m_blk, cnt, h, wrt, b)


def _combine0(m_blk, cnt, wlt, x, wrt, b):
    din = x.shape[1]
    dout = wrt.shape[1]
    nwb = dout // WB

    def body(m_ref, c_ref, wl_ref, x_ref, wr_ref, b_ref, o_ref):
        recip = 1.0 / jnp.maximum(c_ref[...], 1.0)
        mean = m_ref[0] * recip
        acc = jnp.dot(mean, wl_ref[...], preferred_element_type=jnp.float32)
        acc += jnp.dot(x_ref[...], wr_ref[...], preferred_element_type=jnp.float32)
        o_ref[...] = jnp.maximum(acc + b_ref[...], 0.0)

    return pl.pallas_call(
        body,
        grid=(NP // BN, nwb),
        in_specs=[
            pl.BlockSpec((1, BN, din), lambda i, j: (0, i, 0)),
            pl.BlockSpec((BN, 1), lambda i, j: (i, 0)),
            pl.BlockSpec((din, WB), lambda i, j: (0, j)),
            pl.BlockSpec((BN, din), lambda i, j: (i, 0)),
            pl.BlockSpec((din, WB), lambda i, j: (0, j)),
            pl.BlockSpec((1, WB), lambda i, j: (0, j)),
        ],
        out_specs=pl.BlockSpec((BN, WB), lambda i, j: (i, j)),
        out_shape=jax.ShapeDtypeStruct((NP, dout), jnp.float32),
    )(m_blk, cnt, wlt, x, wrt, b)


def _head(h, f1t, b1, f2t, b2):
    def body(h_ref, w1_ref, b1_ref, w2_ref, b2_ref, o_ref):
        z = jnp.maximum(jnp.dot(h_ref[...], w1_ref[...],
                                preferred_element_type=jnp.float32) + b1_ref[...], 0.0)
        o = jnp.dot(z, w2_ref[...], preferred_element_type=jnp.float32) + b2_ref[...]
        valid = lax.broadcasted_iota(jnp.int32, o.shape, 1) < 2
        om = jnp.where(valid, o, -jnp.inf)
        m = jnp.max(om, axis=1, keepdims=True)
        e = jnp.where(valid, jnp.exp(o - m), 0.0)
        lse = m + jnp.log(jnp.sum(e, axis=1, keepdims=True))
        o_ref[...] = o - lse

    return pl.pallas_call(
        body,
        grid=(NP // BN,),
        in_specs=[
            pl.BlockSpec((BN, 128), lambda i: (i, 0)),
            pl.BlockSpec((128, 128), lambda i: (0, 0)),
            pl.BlockSpec((1, 128), lambda i: (0, 0)),
            pl.BlockSpec((128, 128), lambda i: (0, 0)),
            pl.BlockSpec((1, 128), lambda i: (0, 0)),
        ],
        out_specs=pl.BlockSpec((BN, 128), lambda i: (i, 0)),
        out_shape=jax.ShapeDtypeStruct((NP, 128), jnp.float32),
    )(h, f1t, b1, f2t, b2)


def _pad2(a, r, c):
    return jnp.pad(a, ((0, r - a.shape[0]), (0, c - a.shape[1])))


def kernel(x, edge_index, params):
    dims = [128, 1800, 1650, 1500, 1350, 1200, 1050, 900, 750, 600, 450, 300, 150, 64]
    pd = [_pad128(d) for d in dims]

    src = edge_index[0].astype(jnp.int32)
    dst = edge_index[1].astype(jnp.int32)
    zeros = jnp.zeros((NP, WB), jnp.float32)

    xp = jnp.pad(x, ((0, NP - N), (0, 0)))
    ones_blk = jnp.zeros((NP, WB), jnp.float32).at[:, 0].set(1.0)
    x_aug = jnp.stack([xp, ones_blk])

    m_aug = _make_agg(2)(x_aug, src, dst, zeros)
    cnt = m_aug[1, :, 0:1]
    m0 = m_aug[0:1]

    wl0 = _pad2(params["Wl0"].T, pd[0], pd[1])
    wr0 = _pad2(params["Wr0"].T, pd[0], pd[1])
    b0 = _pad2(params["bl0"][None, :], 1, pd[1])
    h = _combine0(m0, cnt, wl0, xp, wr0, b0)

    for i in range(1, 13):
        wlt = _pad2(params[f"Wl{i}"].T, pd[i], pd[i + 1])
        wrt = _pad2(params[f"Wr{i}"].T, pd[i], pd[i + 1])
        b = _pad2(params[f"bl{i}"][None, :], 1, pd[i + 1])
        p_blk = _proj(h, wlt)
        m_blk = _make_agg(pd[i + 1] // WB)(p_blk, src, dst, zeros)
        h = _combine(m_blk, cnt, h, wrt, b)

    f1t = _pad2(params["fc1_W"].T, 128, 128)
    b1 = _pad2(params["fc1_b"][None, :], 1, 128)
    f2t = _pad2(params["fc2_W"].T, 128, 128)
    b2 = _pad2(params["fc2_b"][None, :], 1, 128)
    out = _head(h, f1t, b1, f2t, b2)
    return out[:N, :2]


# speedup vs baseline: 4.2796x; 1.8199x over previous
import functools

import jax
import jax.numpy as jnp
from jax import lax
from jax.experimental import pallas as pl
from jax.experimental.pallas import tpu as pltpu
from jax.experimental.pallas import tpu_sc as plsc

N = 10000
E = 320000
NP = 10240
WB = 128
BN = 256
N_TILES = 16
CHUNK = 128
N_CHUNKS = 160
E_PAD = CHUNK * N_CHUNKS * N_TILES
ROWS_PER_TILE = NP // N_TILES
IDX_SHIFT = 14


def _pad128(d):
    return ((d + 127) // 128) * 128


@functools.lru_cache(maxsize=None)
def _make_agg(nwb):
    mesh = plsc.VectorSubcoreMesh(core_axis_name="c", subcore_axis_name="s")
    n_pairs = N_CHUNKS // 2

    @functools.partial(
        pl.kernel,
        mesh=mesh,
        out_type=jax.ShapeDtypeStruct((nwb, NP, WB), jnp.float32),
        scratch_types=[
            pltpu.VMEM((CHUNK,), jnp.int32),
            pltpu.VMEM((CHUNK,), jnp.int32),
            pltpu.VMEM((CHUNK,), jnp.int32),
            pltpu.VMEM((CHUNK,), jnp.int32),
            pltpu.VMEM((CHUNK,), jnp.int32),
            pltpu.VMEM((CHUNK,), jnp.int32),
            pltpu.VMEM((CHUNK, WB), jnp.float32),
            pltpu.VMEM((CHUNK, WB), jnp.float32),
            pltpu.VMEM_SHARED((NP, WB), jnp.float32),
            pltpu.SemaphoreType.DMA,
            pltpu.SemaphoreType.DMA,
            pltpu.SemaphoreType.DMA,
            pltpu.SemaphoreType.DMA,
        ],
    )
    def agg(p_hbm, pk_hbm, zeros_hbm, out_hbm,
            pkv0, pkv1, sv0, dv0, sv1, dv1, rows0, rows1, acc_sh,
            p0, p1, g0, g1):
        c = lax.axis_index("c")
        s = lax.axis_index("s")
        row0 = s * ROWS_PER_TILE
        ebase = s * (N_CHUNKS * CHUNK)

        def pk_load(k, pkv, sem):
            pltpu.async_copy(pk_hbm.at[pl.ds(ebase + k * CHUNK, CHUNK)],
                             pkv, sem)

        def pk_wait(pkv, sem):
            pltpu.make_async_copy(pk_hbm.at[pl.ds(ebase, CHUNK)],
                                  pkv, sem).wait()

        def unpack(pkv, sv, dv):
            for i in range(CHUNK // 16):
                v = pkv[pl.ds(16 * i, 16)]
                sv[pl.ds(16 * i, 16)] = v & ((1 << IDX_SHIFT) - 1)
                dv[pl.ds(16 * i, 16)] = lax.shift_right_logical(v, IDX_SHIFT)

        def gather(wj, sv, buf, sem):
            pltpu.async_copy(p_hbm.at[wj].at[sv], buf, sem)

        def gwait(wj, sv, buf, sem):
            pltpu.make_async_copy(p_hbm.at[wj].at[sv], buf, sem).wait()

        def scatter(dv, buf):
            pltpu.sync_copy(buf, acc_sh.at[dv], add=True)

        for wj in range(nwb):
            @pl.when(c == (wj % 2))
            def _():
                pltpu.sync_copy(zeros_hbm.at[pl.ds(row0, ROWS_PER_TILE)],
                                acc_sh.at[pl.ds(row0, ROWS_PER_TILE)])
                plsc.subcore_barrier()

                pk_load(0, pkv0, p0)
                pk_wait(pkv0, p0)
                unpack(pkv0, sv0, dv0)
                gather(wj, sv0, rows0, g0)
                pk_load(1, pkv1, p1)

                def pair(k2, carry):
                    ka = 2 * k2
                    last = k2 >= n_pairs - 1
                    pk_wait(pkv1, p1)
                    unpack(pkv1, sv1, dv1)
                    gather(wj, sv1, rows1, g1)

                    @pl.when(jnp.logical_not(last))
                    def _():
                        pk_load(ka + 2, pkv0, p0)

                    gwait(wj, sv0, rows0, g0)
                    scatter(dv0, rows0)

                    @pl.when(jnp.logical_not(last))
                    def _():
                        pk_wait(pkv0, p0)
                        unpack(pkv0, sv0, dv0)
                        gather(wj, sv0, rows0, g0)
                        pk_load(ka + 3, pkv1, p1)

                    gwait(wj, sv1, rows1, g1)
                    scatter(dv1, rows1)
                    return carry

                lax.fori_loop(0, n_pairs, pair, 0)

                plsc.subcore_barrier()
                pltpu.sync_copy(acc_sh.at[pl.ds(row0, ROWS_PER_TILE)],
                                out_hbm.at[wj].at[pl.ds(row0, ROWS_PER_TILE)])
                plsc.subcore_barrier()

    return agg


def _proj(h, wlt):
    din = h.shape[1]
    nwb = wlt.shape[1] // WB

    def body(h_ref, w_ref, o_ref):
        o_ref[...] = jnp.dot(h_ref[...], w_ref[...],
                             preferred_element_type=jnp.float32)[None]

    return pl.pallas_call(
        body,
        grid=(NP // BN, nwb),
        in_specs=[
            pl.BlockSpec((BN, din), lambda i, j: (i, 0)),
            pl.BlockSpec((din, WB), lambda i, j: (0, j)),
        ],
        out_specs=pl.BlockSpec((1, BN, WB), lambda i, j: (j, i, 0)),
        out_shape=jax.ShapeDtypeStruct((nwb, NP, WB), jnp.float32),
    )(h, wlt)


def _combine(m_blk, cnt, h, wrt, b):
    din = h.shape[1]
    dout = wrt.shape[1]
    nwb = dout // WB

    def body(m_ref, c_ref, h_ref, w_ref, b_ref, o_ref):
        recip = 1.0 / jnp.maximum(c_ref[...], 1.0)
        acc = m_ref[0] * recip + b_ref[...]
        acc += jnp.dot(h_ref[...], w_ref[...], preferred_element_type=jnp.float32)
        o_ref[...] = jnp.maximum(acc, 0.0)

    return pl.pallas_call(
        body,
        grid=(NP // BN, nwb),
        in_specs=[
            pl.BlockSpec((1, BN, WB), lambda i, j: (j, i, 0)),
            pl.BlockSpec((BN, 1), lambda i, j: (i, 0)),
            pl.BlockSpec((BN, din), lambda i, j: (i, 0)),
            pl.BlockSpec((din, WB), lambda i, j: (0, j)),
            pl.BlockSpec((1, WB), lambda i, j: (0, j)),
        ],
        out_specs=pl.BlockSpec((BN, WB), lambda i, j: (i, j)),
        out_shape=jax.ShapeDtypeStruct((NP, dout), jnp.float32),
    )(m_blk, cnt, h, wrt, b)


def _combine0(m_blk, cnt, wlt, x, wrt, b):
    din = x.shape[1]
    dout = wrt.shape[1]
    nwb = dout // WB

    def body(m_ref, c_ref, wl_ref, x_ref, wr_ref, b_ref, o_ref):
        recip = 1.0 / jnp.maximum(c_ref[...], 1.0)
        mean = m_ref[0] * recip
        acc = jnp.dot(mean, wl_ref[...], preferred_element_type=jnp.float32)
        acc += jnp.dot(x_ref[...], wr_ref[...], preferred_element_type=jnp.float32)
        o_ref[...] = jnp.maximum(acc + b_ref[...], 0.0)

    return pl.pallas_call(
        body,
        grid=(NP // BN, nwb),
        in_specs=[
            pl.BlockSpec((1, BN, din), lambda i, j: (0, i, 0)),
            pl.BlockSpec((BN, 1), lambda i, j: (i, 0)),
            pl.BlockSpec((din, WB), lambda i, j: (0, j)),
            pl.BlockSpec((BN, din), lambda i, j: (i, 0)),
            pl.BlockSpec((din, WB), lambda i, j: (0, j)),
            pl.BlockSpec((1, WB), lambda i, j: (0, j)),
        ],
        out_specs=pl.BlockSpec((BN, WB), lambda i, j: (i, j)),
        out_shape=jax.ShapeDtypeStruct((NP, dout), jnp.float32),
    )(m_blk, cnt, wlt, x, wrt, b)


def _head(h, f1t, b1, f2t, b2):
    def body(h_ref, w1_ref, b1_ref, w2_ref, b2_ref, o_ref):
        z = jnp.maximum(jnp.dot(h_ref[...], w1_ref[...],
                                preferred_element_type=jnp.float32) + b1_ref[...], 0.0)
        o = jnp.dot(z, w2_ref[...], preferred_element_type=jnp.float32) + b2_ref[...]
        valid = lax.broadcasted_iota(jnp.int32, o.shape, 1) < 2
        om = jnp.where(valid, o, -jnp.inf)
        m = jnp.max(om, axis=1, keepdims=True)
        e = jnp.where(valid, jnp.exp(o - m), 0.0)
        lse = m + jnp.log(jnp.sum(e, axis=1, keepdims=True))
        o_ref[...] = o - lse

    return pl.pallas_call(
        body,
        grid=(NP // BN,),
        in_specs=[
            pl.BlockSpec((BN, 128), lambda i: (i, 0)),
            pl.BlockSpec((128, 128), lambda i: (0, 0)),
            pl.BlockSpec((1, 128), lambda i: (0, 0)),
            pl.BlockSpec((128, 128), lambda i: (0, 0)),
            pl.BlockSpec((1, 128), lambda i: (0, 0)),
        ],
        out_specs=pl.BlockSpec((BN, 128), lambda i: (i, 0)),
        out_shape=jax.ShapeDtypeStruct((NP, 128), jnp.float32),
    )(h, f1t, b1, f2t, b2)


def _pad2(a, r, c):
    return jnp.pad(a, ((0, r - a.shape[0]), (0, c - a.shape[1])))


def kernel(x, edge_index, params):
    dims = [128, 1800, 1650, 1500, 1350, 1200, 1050, 900, 750, 600, 450, 300, 150, 64]
    pd = [_pad128(d) for d in dims]

    n_pad = E_PAD - E
    pad_ar = jnp.arange(n_pad, dtype=jnp.int32)
    src = jnp.concatenate([edge_index[0].astype(jnp.int32), pad_ar % N])
    dst = jnp.concatenate([edge_index[1].astype(jnp.int32),
                           N + pad_ar % (NP - N)])
    packed = src | (dst << IDX_SHIFT)
    zeros = jnp.zeros((NP, WB), jnp.float32)

    xp = jnp.pad(x, ((0, NP - N), (0, 0)))
    ones_blk = jnp.zeros((NP, WB), jnp.float32).at[:, 0].set(1.0)
    x_aug = jnp.stack([xp, ones_blk])

    m_aug = _make_agg(2)(x_aug, packed, zeros)
    cnt = m_aug[1, :, 0:1]
    m0 = m_aug[0:1]

    wl0 = _pad2(params["Wl0"].T, pd[0], pd[1])
    wr0 = _pad2(params["Wr0"].T, pd[0], pd[1])
    b0 = _pad2(params["bl0"][None, :], 1, pd[1])
    h = _combine0(m0, cnt, wl0, xp, wr0, b0)

    for i in range(1, 13):
        wlt = _pad2(params[f"Wl{i}"].T, pd[i], pd[i + 1])
        wrt = _pad2(params[f"Wr{i}"].T, pd[i], pd[i + 1])
        b = _pad2(params[f"bl{i}"][None, :], 1, pd[i + 1])
        p_blk = _proj(h, wlt)
        m_blk = _make_agg(pd[i + 1] // WB)(p_blk, packed, zeros)
        h = _combine(m_blk, cnt, h, wrt, b)

    f1t = _pad2(params["fc1_W"].T, 128, 128)
    b1 = _pad2(params["fc1_b"][None, :], 1, 128)
    f2t = _pad2(params["fc2_W"].T, 128, 128)
    b2 = _pad2(params["fc2_b"][None, :], 1, 128)
    out = _head(h, f1t, b1, f2t, b2)
    return out[:N, :2]


# speedup vs baseline: 6.0884x; 1.4227x over previous
import functools

import jax
import jax.numpy as jnp
from jax import lax
from jax.experimental import pallas as pl
from jax.experimental.pallas import tpu as pltpu
from jax.experimental.pallas import tpu_sc as plsc

N = 10000
E = 320000
NP = 10240
WB = 128
BN = 2048
N_TILES = 16
CHUNK = 128
N_CHUNKS = 160
E_PAD = CHUNK * N_CHUNKS * N_TILES
ROWS_PER_TILE = NP // N_TILES
IDX_SHIFT = 14


def _pad128(d):
    return ((d + 127) // 128) * 128


@functools.lru_cache(maxsize=None)
def _make_agg(nwb):
    mesh = plsc.VectorSubcoreMesh(core_axis_name="c", subcore_axis_name="s")
    n_pairs = N_CHUNKS // 2

    @functools.partial(
        pl.kernel,
        mesh=mesh,
        out_type=jax.ShapeDtypeStruct((nwb, NP, WB), jnp.float32),
        scratch_types=[
            pltpu.VMEM((CHUNK,), jnp.int32),
            pltpu.VMEM((CHUNK,), jnp.int32),
            pltpu.VMEM((CHUNK,), jnp.int32),
            pltpu.VMEM((CHUNK,), jnp.int32),
            pltpu.VMEM((CHUNK,), jnp.int32),
            pltpu.VMEM((CHUNK,), jnp.int32),
            pltpu.VMEM((CHUNK, WB), jnp.float32),
            pltpu.VMEM((CHUNK, WB), jnp.float32),
            pltpu.VMEM_SHARED((NP, WB), jnp.float32),
            pltpu.SemaphoreType.DMA,
            pltpu.SemaphoreType.DMA,
            pltpu.SemaphoreType.DMA,
            pltpu.SemaphoreType.DMA,
        ],
    )
    def agg(p_hbm, pk_hbm, zeros_hbm, out_hbm,
            pkv0, pkv1, sv0, dv0, sv1, dv1, rows0, rows1, acc_sh,
            p0, p1, g0, g1):
        c = lax.axis_index("c")
        s = lax.axis_index("s")
        row0 = s * ROWS_PER_TILE
        ebase = s * (N_CHUNKS * CHUNK)

        def pk_load(k, pkv, sem):
            pltpu.async_copy(pk_hbm.at[pl.ds(ebase + k * CHUNK, CHUNK)],
                             pkv, sem)

        def pk_wait(pkv, sem):
            pltpu.make_async_copy(pk_hbm.at[pl.ds(ebase, CHUNK)],
                                  pkv, sem).wait()

        def unpack(pkv, sv, dv):
            for i in range(CHUNK // 16):
                v = pkv[pl.ds(16 * i, 16)]
                sv[pl.ds(16 * i, 16)] = v & ((1 << IDX_SHIFT) - 1)
                dv[pl.ds(16 * i, 16)] = lax.shift_right_logical(v, IDX_SHIFT)

        def gather(wj, sv, buf, sem):
            pltpu.async_copy(p_hbm.at[wj].at[sv], buf, sem)

        def gwait(wj, sv, buf, sem):
            pltpu.make_async_copy(p_hbm.at[wj].at[sv], buf, sem).wait()

        def scatter(dv, buf):
            pltpu.sync_copy(buf, acc_sh.at[dv], add=True)

        for wj in range(nwb):
            @pl.when(c == (wj % 2))
            def _():
                pltpu.sync_copy(zeros_hbm.at[pl.ds(row0, ROWS_PER_TILE)],
                                acc_sh.at[pl.ds(row0, ROWS_PER_TILE)])
                plsc.subcore_barrier()

                pk_load(0, pkv0, p0)
                pk_wait(pkv0, p0)
                unpack(pkv0, sv0, dv0)
                gather(wj, sv0, rows0, g0)
                pk_load(1, pkv1, p1)

                def pair(k2, carry):
                    ka = 2 * k2
                    last = k2 >= n_pairs - 1
                    pk_wait(pkv1, p1)
                    unpack(pkv1, sv1, dv1)
                    gather(wj, sv1, rows1, g1)

                    @pl.when(jnp.logical_not(last))
                    def _():
                        pk_load(ka + 2, pkv0, p0)

                    gwait(wj, sv0, rows0, g0)
                    scatter(dv0, rows0)

                    @pl.when(jnp.logical_not(last))
                    def _():
                        pk_wait(pkv0, p0)
                        unpack(pkv0, sv0, dv0)
                        gather(wj, sv0, rows0, g0)
                        pk_load(ka + 3, pkv1, p1)

                    gwait(wj, sv1, rows1, g1)
                    scatter(dv1, rows1)
                    return carry

                lax.fori_loop(0, n_pairs, pair, 0)

                plsc.subcore_barrier()
                pltpu.sync_copy(acc_sh.at[pl.ds(row0, ROWS_PER_TILE)],
                                out_hbm.at[wj].at[pl.ds(row0, ROWS_PER_TILE)])
                plsc.subcore_barrier()

    return agg


def _proj(h, wlt):
    din = h.shape[1]
    nwb = wlt.shape[1] // WB

    def body(h_ref, w_ref, o_ref):
        o_ref[...] = jnp.dot(h_ref[...], w_ref[...],
                             preferred_element_type=jnp.float32)[None]

    return pl.pallas_call(
        body,
        grid=(NP // BN, nwb),
        in_specs=[
            pl.BlockSpec((BN, din), lambda i, j: (i, 0)),
            pl.BlockSpec((din, WB), lambda i, j: (0, j)),
        ],
        out_specs=pl.BlockSpec((1, BN, WB), lambda i, j: (j, i, 0)),
        out_shape=jax.ShapeDtypeStruct((nwb, NP, WB), jnp.float32),
    )(h, wlt)


def _rpart(h, wrt, b):
    din = h.shape[1]
    dout = wrt.shape[1]
    nwb = dout // WB

    def body(h_ref, w_ref, b_ref, o_ref):
        o_ref[...] = jnp.dot(h_ref[...], w_ref[...],
                             preferred_element_type=jnp.float32) + b_ref[...]

    return pl.pallas_call(
        body,
        grid=(NP // BN, nwb),
        in_specs=[
            pl.BlockSpec((BN, din), lambda i, j: (i, 0)),
            pl.BlockSpec((din, WB), lambda i, j: (0, j)),
            pl.BlockSpec((1, WB), lambda i, j: (0, j)),
        ],
        out_specs=pl.BlockSpec((BN, WB), lambda i, j: (i, j)),
        out_shape=jax.ShapeDtypeStruct((NP, dout), jnp.float32),
    )(h, wrt, b)


def _fuse(m_blk, cnt, r):
    dout = r.shape[1]
    nwb = dout // WB

    def body(m_ref, c_ref, r_ref, o_ref):
        recip = 1.0 / jnp.maximum(c_ref[...], 1.0)
        o_ref[...] = jnp.maximum(m_ref[0] * recip + r_ref[...], 0.0)

    return pl.pallas_call(
        body,
        grid=(NP // BN, nwb),
        in_specs=[
            pl.BlockSpec((1, BN, WB), lambda i, j: (j, i, 0)),
            pl.BlockSpec((BN, 1), lambda i, j: (i, 0)),
            pl.BlockSpec((BN, WB), lambda i, j: (i, j)),
        ],
        out_specs=pl.BlockSpec((BN, WB), lambda i, j: (i, j)),
        out_shape=jax.ShapeDtypeStruct((NP, dout), jnp.float32),
    )(m_blk, cnt, r)


def _fuse0(m_blk, cnt, wlt, r):
    din = m_blk.shape[2]
    dout = r.shape[1]
    nwb = dout // WB

    def body(m_ref, c_ref, wl_ref, r_ref, o_ref):
        recip = 1.0 / jnp.maximum(c_ref[...], 1.0)
        mean = m_ref[0] * recip
        acc = jnp.dot(mean, wl_ref[...], preferred_element_type=jnp.float32)
        o_ref[...] = jnp.maximum(acc + r_ref[...], 0.0)

    return pl.pallas_call(
        body,
        grid=(NP // BN, nwb),
        in_specs=[
            pl.BlockSpec((1, BN, din), lambda i, j: (0, i, 0)),
            pl.BlockSpec((BN, 1), lambda i, j: (i, 0)),
            pl.BlockSpec((din, WB), lambda i, j: (0, j)),
            pl.BlockSpec((BN, WB), lambda i, j: (i, j)),
        ],
        out_specs=pl.BlockSpec((BN, WB), lambda i, j: (i, j)),
        out_shape=jax.ShapeDtypeStruct((NP, dout), jnp.float32),
    )(m_blk, cnt, wlt, r)


def _head(h, f1t, b1, f2t, b2):
    def body(h_ref, w1_ref, b1_ref, w2_ref, b2_ref, o_ref):
        z = jnp.maximum(jnp.dot(h_ref[...], w1_ref[...],
                                preferred_element_type=jnp.float32) + b1_ref[...], 0.0)
        o = jnp.dot(z, w2_ref[...], preferred_element_type=jnp.float32) + b2_ref[...]
        valid = lax.broadcasted_iota(jnp.int32, o.shape, 1) < 2
        om = jnp.where(valid, o, -jnp.inf)
        m = jnp.max(om, axis=1, keepdims=True)
        e = jnp.where(valid, jnp.exp(o - m), 0.0)
        lse = m + jnp.log(jnp.sum(e, axis=1, keepdims=True))
        o_ref[...] = o - lse

    return pl.pallas_call(
        body,
        grid=(NP // BN,),
        in_specs=[
            pl.BlockSpec((BN, 128), lambda i: (i, 0)),
            pl.BlockSpec((128, 128), lambda i: (0, 0)),
            pl.BlockSpec((1, 128), lambda i: (0, 0)),
            pl.BlockSpec((128, 128), lambda i: (0, 0)),
            pl.BlockSpec((1, 128), lambda i: (0, 0)),
        ],
        out_specs=pl.BlockSpec((BN, 128), lambda i: (i, 0)),
        out_shape=jax.ShapeDtypeStruct((NP, 128), jnp.float32),
    )(h, f1t, b1, f2t, b2)


def _pad2(a, r, c):
    return jnp.pad(a, ((0, r - a.shape[0]), (0, c - a.shape[1])))


def kernel(x, edge_index, params):
    dims = [128, 1800, 1650, 1500, 1350, 1200, 1050, 900, 750, 600, 450, 300, 150, 64]
    pd = [_pad128(d) for d in dims]

    n_pad = E_PAD - E
    pad_ar = jnp.arange(n_pad, dtype=jnp.int32)
    src = jnp.concatenate([edge_index[0].astype(jnp.int32), pad_ar % N])
    dst = jnp.concatenate([edge_index[1].astype(jnp.int32),
                           N + pad_ar % (NP - N)])
    packed = src | (dst << IDX_SHIFT)
    zeros = jnp.zeros((NP, WB), jnp.float32)

    xp = jnp.pad(x, ((0, NP - N), (0, 0)))
    ones_blk = jnp.zeros((NP, WB), jnp.float32).at[:, 0].set(1.0)
    x_aug = jnp.stack([xp, ones_blk])

    m_aug = _make_agg(2)(x_aug, packed, zeros)
    cnt = m_aug[1, :, 0:1]
    m0 = m_aug[0:1]

    wl0 = _pad2(params["Wl0"].T, pd[0], pd[1])
    wr0 = _pad2(params["Wr0"].T, pd[0], pd[1])
    b0 = _pad2(params["bl0"][None, :], 1, pd[1])
    r0 = _rpart(xp, wr0, b0)
    h = _fuse0(m0, cnt, wl0, r0)

    for i in range(1, 13):
        wlt = _pad2(params[f"Wl{i}"].T, pd[i], pd[i + 1])
        wrt = _pad2(params[f"Wr{i}"].T, pd[i], pd[i + 1])
        b = _pad2(params[f"bl{i}"][None, :], 1, pd[i + 1])
        p_blk = _proj(h, wlt)
        m_blk = _make_agg(pd[i + 1] // WB)(p_blk, packed, zeros)
        r = _rpart(h, wrt, b)
        h = _fuse(m_blk, cnt, r)

    f1t = _pad2(params["fc1_W"].T, 128, 128)
    b1 = _pad2(params["fc1_b"][None, :], 1, 128)
    f2t = _pad2(params["fc2_W"].T, 128, 128)
    b2 = _pad2(params["fc2_b"][None, :], 1, 128)
    out = _head(h, f1t, b1, f2t, b2)
    return out[:N, :2]
